# baseline (speedup 1.0000x reference)
import functools

import jax
import jax.numpy as jnp
from jax import lax
from jax.experimental import pallas as pl
from jax.experimental.pallas import tpu as pltpu
from jax.experimental.pallas import tpu_sc as plsc

N = 10000
D = 128
DA = 144
E = 320000

NC = 2
NS = 16
NW = NC * NS
C = 80
BLK = 8
NBLKS = 16
CHUNKS = BLK * NBLKS
E_PAD = NW * CHUNKS * C
NPAD = 10112
ROWS_PER_TILE = NPAD // NS


def _prep_edges(edge_index):
    src = edge_index[0].astype(jnp.int32)
    dst = edge_index[1].astype(jnp.int32)
    pad = E_PAD - E
    src = jnp.concatenate([src, jnp.zeros((pad,), jnp.int32)])
    dst = jnp.concatenate([dst, jnp.full((pad,), N, jnp.int32)])
    shape = (NW, NBLKS, BLK, C)
    return src.reshape(shape), dst.reshape(shape)


def _aggregate_sc(h_aug, src_idx, dst_idx, zrows):
    mesh = plsc.VectorSubcoreMesh(core_axis_name="c", subcore_axis_name="s")

    @functools.partial(
        pl.kernel,
        out_type=jax.ShapeDtypeStruct((NC, NPAD, DA), jnp.float32),
        mesh=mesh,
        compiler_params=pltpu.CompilerParams(use_tc_tiling_on_sc=False),
        scratch_types=[
            pltpu.VMEM((C, DA), jnp.float32),
            pltpu.VMEM((C, DA), jnp.float32),
            pltpu.VMEM((4, BLK, C), jnp.int32),
            pltpu.VMEM_SHARED((NPAD, DA), jnp.float32),
            pltpu.SemaphoreType.DMA,
            pltpu.SemaphoreType.DMA,
            pltpu.SemaphoreType.DMA,
            pltpu.SemaphoreType.DMA,
        ],
    )
    def k(h_hbm, src_hbm, dst_hbm, zrows_hbm, agg_hbm,
          buf0, buf1, idx_v, agg_sh, sem0, sem1, semIA, semIB):
        srcA, dstA = idx_v.at[0], idx_v.at[1]
        srcB, dstB = idx_v.at[2], idx_v.at[3]
        cid = lax.axis_index("c")
        sid = lax.axis_index("s")
        wid = cid * NS + sid
        bufs = (buf0, buf1)
        sems = (sem0, sem1)

        def fire(src_row, buf, sem):
            pltpu.async_copy(h_hbm.at[src_row], buf, sem)

        def drain(src_row, buf, sem):
            pltpu.make_async_copy(h_hbm.at[src_row], buf, sem).wait()

        def scatter(dst_row, buf):
            pltpu.sync_copy(buf, agg_sh.at[dst_row], add=True)

        def fetch_idx(blk, sref, dref, sem):
            pltpu.async_copy(src_hbm.at[wid, blk], sref, sem)
            pltpu.async_copy(dst_hbm.at[wid, blk], dref, sem)

        def wait_idx(blk, sref, dref, sem):
            pltpu.make_async_copy(src_hbm.at[wid, blk], sref, sem).wait()
            pltpu.make_async_copy(dst_hbm.at[wid, blk], dref, sem).wait()

        pltpu.sync_copy(src_hbm.at[wid, 0], srcA)
        pltpu.sync_copy(dst_hbm.at[wid, 0], dstA)
        fire(srcA.at[0], buf0, sem0)
        fire(srcA.at[1], buf1, sem1)
        fetch_idx(1, srcB, dstB, semIB)

        base = sid * ROWS_PER_TILE
        pltpu.sync_copy(zrows_hbm, agg_sh.at[pl.ds(base, ROWS_PER_TILE)])

        plsc.subcore_barrier()

        def block_body(bb, cur_src, cur_dst, nxt_src, nxt_dst,
                       semI_nxt, semI_cur, guard_next):
            for jj in range(BLK):
                p = jj % 2
                drain(cur_src.at[jj], bufs[p], sems[p])
                scatter(cur_dst.at[jj], bufs[p])
                if jj < BLK - 2:
                    fire(cur_src.at[jj + 2], bufs[p], sems[p])
                elif jj == BLK - 2:
                    def head0():
                        wait_idx(bb + 1, nxt_src, nxt_dst, semI_nxt)
                        fire(nxt_src.at[0], bufs[p], sems[p])
                    if guard_next is None:
                        head0()
                    else:
                        pl.when(guard_next)(head0)
                else:
                    def tail():
                        fire(nxt_src.at[1], bufs[p], sems[p])
                    if guard_next is None:
                        tail()
                    else:
                        pl.when(guard_next)(tail)

                    @pl.when(bb + 2 < NBLKS)
                    def _():
                        fetch_idx(bb + 2, cur_src, cur_dst, semI_cur)

        @pl.loop(0, NBLKS, step=2)
        def _(b):
            block_body(b, srcA, dstA, srcB, dstB, semIB, semIA, None)
            block_body(b + 1, srcB, dstB, srcA, dstA, semIA, semIB,
                       b + 2 < NBLKS)

        plsc.subcore_barrier()
        pltpu.sync_copy(agg_sh.at[pl.ds(base, ROWS_PER_TILE)],
                        agg_hbm.at[cid, pl.ds(base, ROWS_PER_TILE)])

    return k(h_aug, src_idx, dst_idx, zrows)


def _dense_tc(hx_aug, agg, W_self, W_neigh, b, relu):
    R = 2000
    out_w = DA if relu else D

    def body(h_ref, agg_ref, ws_ref, wn_ref, b_ref, o_ref):
        aggs = agg_ref[0] + agg_ref[1]
        deg = jnp.maximum(aggs[:, 128:129], 1.0)
        hn = aggs[:, :D] / deg
        acc = (
            jnp.dot(h_ref[:, :D], ws_ref[...],
                    preferred_element_type=jnp.float32,
                    precision=lax.Precision.HIGHEST)
            + jnp.dot(hn, wn_ref[...],
                      preferred_element_type=jnp.float32,
                      precision=lax.Precision.HIGHEST)
            + b_ref[...]
        )
        if relu:
            acc = jnp.maximum(acc, 0.0)
            aug = jnp.concatenate(
                [jnp.ones((R, 1), jnp.float32),
                 jnp.zeros((R, DA - D - 1), jnp.float32)], axis=1)
            o_ref[...] = jnp.concatenate([acc, aug], axis=1)
        else:
            o_ref[...] = acc

    return pl.pallas_call(
        body,
        grid=(N // R,),
        in_specs=[
            pl.BlockSpec((R, DA), lambda i: (i, 0)),
            pl.BlockSpec((NC, R, DA), lambda i: (0, i, 0)),
            pl.BlockSpec((D, D), lambda i: (0, 0)),
            pl.BlockSpec((D, D), lambda i: (0, 0)),
            pl.BlockSpec((1, D), lambda i: (0, 0)),
        ],
        out_specs=pl.BlockSpec((R, out_w), lambda i: (i, 0)),
        out_shape=jax.ShapeDtypeStruct((N, out_w), jnp.float32),
    )(hx_aug, agg, W_self, W_neigh, b.reshape(1, D))


def kernel(h, edge_index0, edge_index1, W_self0, W_neigh0, b0,
           W_self1, W_neigh1, b1):
    src0, dst0 = _prep_edges(edge_index0)
    src1, dst1 = _prep_edges(edge_index1)
    zrows = jnp.zeros((ROWS_PER_TILE, DA), jnp.float32)
    h_aug = jnp.concatenate(
        [h, jnp.ones((N, 1), jnp.float32), jnp.zeros((N, DA - D - 1), jnp.float32)],
        axis=1)
    agg0 = _aggregate_sc(h_aug, src0, dst0, zrows)
    x_aug = _dense_tc(h_aug, agg0, W_self0, W_neigh0, b0, relu=True)
    agg1 = _aggregate_sc(x_aug, src1, dst1, zrows)
    return _dense_tc(x_aug, agg1, W_self1, W_neigh1, b1, relu=False)


# speedup vs baseline: 1.5535x; 1.5535x over previous
import functools

import numpy as np

import jax
import jax.numpy as jnp
from jax import lax
from jax.experimental import pallas as pl
from jax.experimental.pallas import tpu as pltpu
from jax.experimental.pallas import tpu_sc as plsc

N = 10000
D = 128
E = 320000

NC = 2
NS = 16
NW = NC * NS
C = 80
BLK = 8
NBLKS = 16
CHUNKS = BLK * NBLKS
E_PAD = NW * CHUNKS * C
NPAD = 10112
ROWS_PER_TILE = NPAD // NS
DEGW = 16

CHUNK_BYTES = C * D * 4
IDX_BYTES = BLK * C * 4


def _prep_edges(edge_index):
    src = edge_index[0].astype(jnp.int32)
    dst = edge_index[1].astype(jnp.int32)
    pad = E_PAD - E
    src = jnp.concatenate([src, jnp.zeros((pad,), jnp.int32)])
    dst = jnp.concatenate([dst, jnp.full((pad,), N, jnp.int32)])
    shape = (NW, NBLKS, BLK, C)
    return src.reshape(shape), dst.reshape(shape)


def _aggregate_sc(h, src_idx, dst_idx, zrows, zdeg, ones):
    mesh = plsc.VectorSubcoreMesh(core_axis_name="c", subcore_axis_name="s")

    @functools.partial(
        pl.kernel,
        out_type=(
            jax.ShapeDtypeStruct((NC, NPAD, D), jnp.float32),
            jax.ShapeDtypeStruct((NC, NPAD, DEGW), jnp.float32),
        ),
        mesh=mesh,
        compiler_params=pltpu.CompilerParams(use_tc_tiling_on_sc=False, needs_layout_passes=False),
        scratch_types=[
            pltpu.VMEM((C, D), jnp.bfloat16),
            pltpu.VMEM((C, D), jnp.bfloat16),
            pltpu.VMEM((C, D), jnp.float32),
            pltpu.VMEM((4, BLK, C), jnp.int32),
            pltpu.VMEM((C, DEGW), jnp.float32),
            pltpu.VMEM_SHARED((NPAD, D), jnp.float32),
            pltpu.VMEM_SHARED((NPAD, DEGW), jnp.float32),
            pltpu.SemaphoreType.DMA,
            pltpu.SemaphoreType.DMA,
            pltpu.SemaphoreType.DMA,
            pltpu.SemaphoreType.DMA,
        ],
    )
    def k(h_hbm, src_hbm, dst_hbm, zrows_hbm, zdeg_hbm, ones_hbm, agg_hbm, deg_hbm,
          buf0, buf1, sbuf, idx_v, ones_v,
          agg_sh, deg_sh, sem0, sem1, semIA, semIB):
        srcA, dstA = idx_v.at[0], idx_v.at[1]
        srcB, dstB = idx_v.at[2], idx_v.at[3]
        cid = lax.axis_index("c")
        sid = lax.axis_index("s")
        wid = cid * NS + sid
        bufs = (buf0, buf1)
        sems = (sem0, sem1)

        def fire(src_row, buf, sem):
            pltpu.async_copy(h_hbm.at[src_row], buf, sem)

        def drain(src_row, buf, sem):
            pltpu.make_async_copy(h_hbm.at[src_row], buf, sem).wait()

        def scatter(dst_row, buf):
            @pl.loop(0, C)
            def _(r):
                for g in range(D // 32):
                    w = plsc.bitcast(buf[r, pl.ds(32 * g, 32)], jnp.int32)
                    ev = plsc.bitcast(lax.shift_left(w, 16), jnp.float32)
                    od = plsc.bitcast(
                        lax.bitwise_and(w, jnp.int32(-65536)), jnp.float32)
                    sbuf[r, pl.ds(32 * g, 16)] = ev
                    sbuf[r, pl.ds(32 * g + 16, 16)] = od

            pltpu.sync_copy(sbuf, agg_sh.at[dst_row], add=True)
            pltpu.sync_copy(ones_v, deg_sh.at[dst_row], add=True)

        def fetch_idx(blk, sref, dref, sem):
            pltpu.async_copy(src_hbm.at[wid, blk], sref, sem)
            pltpu.async_copy(dst_hbm.at[wid, blk], dref, sem)

        def wait_idx(blk, sref, dref, sem):
            pltpu.make_async_copy(src_hbm.at[wid, blk], sref, sem).wait()
            pltpu.make_async_copy(dst_hbm.at[wid, blk], dref, sem).wait()

        pltpu.sync_copy(src_hbm.at[wid, 0], srcA)
        pltpu.sync_copy(dst_hbm.at[wid, 0], dstA)
        fire(srcA.at[0], buf0, sem0)
        fire(srcA.at[1], buf1, sem1)
        fetch_idx(1, srcB, dstB, semIB)

        base = sid * ROWS_PER_TILE
        pltpu.sync_copy(zrows_hbm, agg_sh.at[pl.ds(base, ROWS_PER_TILE)])
        pltpu.sync_copy(zdeg_hbm, deg_sh.at[pl.ds(base, ROWS_PER_TILE)])
        pltpu.sync_copy(ones_hbm, ones_v)

        plsc.subcore_barrier()

        def block_body(bb, cur_src, cur_dst, nxt_src, nxt_dst,
                       semI_nxt, semI_cur, guard_next):
            for jj in range(BLK):
                p = jj % 2
                drain(cur_src.at[jj], bufs[p], sems[p])
                scatter(cur_dst.at[jj], bufs[p])
                if jj < BLK - 2:
                    fire(cur_src.at[jj + 2], bufs[p], sems[p])
                elif jj == BLK - 2:
                    def head0():
                        wait_idx(bb + 1, nxt_src, nxt_dst, semI_nxt)
                        fire(nxt_src.at[0], bufs[p], sems[p])
                    if guard_next is None:
                        head0()
                    else:
                        pl.when(guard_next)(head0)
                else:
                    def tail():
                        fire(nxt_src.at[1], bufs[p], sems[p])
                    if guard_next is None:
                        tail()
                    else:
                        pl.when(guard_next)(tail)

                    @pl.when(bb + 2 < NBLKS)
                    def _():
                        fetch_idx(bb + 2, cur_src, cur_dst, semI_cur)

        @pl.loop(0, NBLKS, step=2)
        def _(b):
            block_body(b, srcA, dstA, srcB, dstB, semIB, semIA, None)
            block_body(b + 1, srcB, dstB, srcA, dstA, semIA, semIB,
                       b + 2 < NBLKS)

        plsc.subcore_barrier()
        pltpu.sync_copy(agg_sh.at[pl.ds(base, ROWS_PER_TILE)],
                        agg_hbm.at[cid, pl.ds(base, ROWS_PER_TILE)])
        pltpu.sync_copy(deg_sh.at[pl.ds(base, ROWS_PER_TILE)],
                        deg_hbm.at[cid, pl.ds(base, ROWS_PER_TILE)])

    return k(h, src_idx, dst_idx, zrows, zdeg, ones)


def _dense_tc(hx, agg, deg, W_self, W_neigh, b, relu):
    R = 2000

    def body(h_ref, agg_ref, deg_ref, ws_ref, wn_ref, b_ref, *outs):
        aggs = agg_ref[0] + agg_ref[1]
        degs = deg_ref[0, :, 0:1] + deg_ref[1, :, 0:1]
        hn = aggs / jnp.maximum(degs, 1.0)
        acc = (
            jnp.dot(h_ref[...], ws_ref[...],
                    preferred_element_type=jnp.float32,
                    precision=lax.Precision.HIGHEST)
            + jnp.dot(hn, wn_ref[...],
                      preferred_element_type=jnp.float32,
                      precision=lax.Precision.HIGHEST)
            + b_ref[...]
        )
        if relu:
            acc = jnp.maximum(acc, 0.0)
            outs[0][...] = acc
            outs[1][...] = acc.astype(jnp.bfloat16)
        else:
            outs[0][...] = acc

    if relu:
        out_specs = [pl.BlockSpec((R, D), lambda i: (i, 0)),
                     pl.BlockSpec((R, D), lambda i: (i, 0))]
        out_shape = [jax.ShapeDtypeStruct((N, D), jnp.float32),
                     jax.ShapeDtypeStruct((N, D), jnp.bfloat16)]
    else:
        out_specs = pl.BlockSpec((R, D), lambda i: (i, 0))
        out_shape = jax.ShapeDtypeStruct((N, D), jnp.float32)

    return pl.pallas_call(
        body,
        grid=(N // R,),
        in_specs=[
            pl.BlockSpec((R, D), lambda i: (i, 0)),
            pl.BlockSpec((NC, R, D), lambda i: (0, i, 0)),
            pl.BlockSpec((NC, R, DEGW), lambda i: (0, i, 0)),
            pl.BlockSpec((D, D), lambda i: (0, 0)),
            pl.BlockSpec((D, D), lambda i: (0, 0)),
            pl.BlockSpec((1, D), lambda i: (0, 0)),
        ],
        out_specs=out_specs,
        out_shape=out_shape,
    )(hx, agg, deg, W_self, W_neigh, b.reshape(1, D))


_MU = np.concatenate([
    np.concatenate([np.arange(32 * g, 32 * g + 32, 2),
                    np.arange(32 * g + 1, 32 * g + 32, 2)])
    for g in range(D // 32)
])


def kernel(h, edge_index0, edge_index1, W_self0, W_neigh0, b0,
           W_self1, W_neigh1, b1):
    src0, dst0 = _prep_edges(edge_index0)
    src1, dst1 = _prep_edges(edge_index1)
    zrows = jnp.zeros((ROWS_PER_TILE, D), jnp.float32)
    zdeg = jnp.zeros((ROWS_PER_TILE, DEGW), jnp.float32)
    ones = jnp.ones((C, DEGW), jnp.float32)
    agg0, deg0 = _aggregate_sc(h.astype(jnp.bfloat16), src0, dst0,
                               zrows, zdeg, ones)
    x, x16 = _dense_tc(h, agg0, deg0, W_self0, W_neigh0[_MU], b0, relu=True)
    agg1, deg1 = _aggregate_sc(x16, src1, dst1, zrows, zdeg, ones)
    return _dense_tc(x, agg1, deg1, W_self1, W_neigh1[_MU], b1, relu=False)
